# baseline (speedup 1.0000x reference)
import functools

import jax
import jax.numpy as jnp
from jax.experimental import pallas as pl
from jax.experimental.pallas import tpu as pltpu

_N_C, _N_L, _D = 10000, 4096, 128
_K_BLK = 1000
_K_STEPS = _N_C // _K_BLK


def _fused_body(adj_ref, xc_ref, xl_ref, c0_ref, wmsg_ref, wflip_ref,
                whh_ref, bias_ref, h_ref, c_ref, acc_ref):
    k = pl.program_id(0)

    @pl.when(k == 0)
    def _init():
        acc_ref[...] = jnp.zeros_like(acc_ref)

    acc_ref[...] += jax.lax.dot_general(
        adj_ref[...], xc_ref[...],
        dimension_numbers=(((0,), (0,)), ((), ())),
        preferred_element_type=jnp.float32)

    @pl.when(k == _K_STEPS - 1)
    def _finish():
        msg = acc_ref[...]
        xl = xl_ref[...]
        n_vars = _N_L // 2
        flipped = jnp.concatenate([xl[n_vars:], xl[:n_vars]], axis=0)

        def mm(a, b):
            return jax.lax.dot_general(
                a, b, dimension_numbers=(((1,), (0,)), ((), ())),
                preferred_element_type=jnp.float32)

        gates = (mm(msg, wmsg_ref[...]) + mm(flipped, wflip_ref[...])
                 + mm(xl, whh_ref[...]) + bias_ref[...])
        i = jax.nn.sigmoid(gates[:, :_D])
        f = jax.nn.sigmoid(gates[:, _D:2 * _D])
        g = jnp.tanh(gates[:, 2 * _D:3 * _D])
        o = jax.nn.sigmoid(gates[:, 3 * _D:])
        c = f * c0_ref[...] + i * g
        h_ref[...] = o * jnp.tanh(c)
        c_ref[...] = c


@functools.partial(jax.jit, static_argnames=())
def kernel(adj_t, x_c, hidden, l_batch, W_ih, W_hh, b_ih, b_hh):
    del l_batch
    x_l = hidden[0]
    c0 = hidden[1]
    wih_t = W_ih.T
    w_msg = wih_t[:_D]
    w_flip = wih_t[_D:]
    whh_t = W_hh.T
    bias = (b_ih + b_hh)[None, :]

    full = lambda shape: pl.BlockSpec(shape, lambda k: (0, 0))
    h, c = pl.pallas_call(
        _fused_body,
        grid=(_K_STEPS,),
        in_specs=[
            pl.BlockSpec((_K_BLK, _N_L), lambda k: (k, 0)),
            pl.BlockSpec((_K_BLK, _D), lambda k: (k, 0)),
            full((_N_L, _D)),
            full((_N_L, _D)),
            full((_D, 4 * _D)),
            full((_D, 4 * _D)),
            full((_D, 4 * _D)),
            full((1, 4 * _D)),
        ],
        out_specs=[full((_N_L, _D)), full((_N_L, _D))],
        out_shape=[jax.ShapeDtypeStruct((_N_L, _D), jnp.float32)] * 2,
        scratch_shapes=[pltpu.VMEM((_N_L, _D), jnp.float32)],
        compiler_params=pltpu.CompilerParams(
            dimension_semantics=("arbitrary",)),
    )(adj_t, x_c, x_l, c0, w_msg, w_flip, whh_t, bias)
    return (h, c)
